# baseline (speedup 1.0000x reference)
import functools

import jax
import jax.numpy as jnp
from jax import lax
from jax.experimental import pallas as pl
from jax.experimental.pallas import tpu as pltpu
from jax.experimental.pallas import tpu_sc as plsc

N_LATENTS = 64
BATCH = 16384
VOCAB = 10
EMB_DIM = 50
OUT_DIM = 2 * N_LATENTS
EPS = 1e-5


def _table_body(x_ref, emb_ref, gamma_ref, beta_ref, w_ref, b_ref, t_ref):
    xv = x_ref[...]
    iota = lax.broadcasted_iota(jnp.int32, (VOCAB, BATCH), 0)
    onehot = (xv[None, :] == iota).astype(jnp.float32)
    counts = jnp.sum(onehot, axis=1).reshape(1, VOCAB)

    emb = emb_ref[...]
    mean = lax.dot_general(counts, emb, (((1,), (0,)), ((), ())),
                           precision=lax.Precision.HIGHEST) * (1.0 / BATCH)
    centered = emb - mean
    var = lax.dot_general(counts, centered * centered, (((1,), (0,)), ((), ())),
                          precision=lax.Precision.HIGHEST) * (1.0 / BATCH)
    inv = 1.0 / jnp.sqrt(var + EPS)
    tbl = jnp.maximum(centered * (inv * gamma_ref[...]) + beta_ref[...], 0.0)
    y = lax.dot_general(tbl, w_ref[...], (((1,), (1,)), ((), ())),
                        precision=lax.Precision.HIGHEST) + b_ref[...]
    t_ref[...] = y


_table_call = pl.pallas_call(
    _table_body,
    out_shape=[jax.ShapeDtypeStruct((VOCAB, OUT_DIM), jnp.float32)],
)


def _split_body(y_ref, o1_ref, o2_ref):
    yv = y_ref[...]
    o1_ref[...] = yv[:, :N_LATENTS]
    o2_ref[...] = yv[:, N_LATENTS:]


_SPLIT_BLK = 4096
_split_call = pl.pallas_call(
    _split_body,
    grid=(BATCH // _SPLIT_BLK,),
    in_specs=[pl.BlockSpec((_SPLIT_BLK, OUT_DIM), lambda i: (i, 0))],
    out_specs=[
        pl.BlockSpec((_SPLIT_BLK, N_LATENTS), lambda i: (i, 0)),
        pl.BlockSpec((_SPLIT_BLK, N_LATENTS), lambda i: (i, 0)),
    ],
    out_shape=[
        jax.ShapeDtypeStruct((BATCH, N_LATENTS), jnp.float32),
        jax.ShapeDtypeStruct((BATCH, N_LATENTS), jnp.float32),
    ],
)


def _make_expand():
    info = plsc.get_sparse_core_info()
    nc, ns = info.num_cores, info.num_subcores
    nw = nc * ns
    rows_per_w = BATCH // nw
    n_chunks = rows_per_w // 128

    mesh = plsc.VectorSubcoreMesh(core_axis_name="c", subcore_axis_name="s")

    @functools.partial(
        pl.kernel,
        mesh=mesh,
        out_type=[
            jax.ShapeDtypeStruct((BATCH // 2, OUT_DIM), jnp.float32),
            jax.ShapeDtypeStruct((BATCH // 2, OUT_DIM), jnp.float32),
        ],
        scratch_types=[
            pltpu.VMEM((rows_per_w,), jnp.int32),
            pltpu.VMEM((VOCAB, OUT_DIM), jnp.float32),
            pltpu.VMEM((n_chunks, 64, OUT_DIM), jnp.float32),
            pltpu.VMEM((n_chunks, 64, OUT_DIM), jnp.float32),
            pltpu.SemaphoreType.DMA,
        ],
        compiler_params=pltpu.CompilerParams(use_tc_tiling_on_sc=False),
    )
    def expand(idx_hbm, t_hbm, out1_hbm, out2_hbm,
               idx_v, table_v, rows1_v, rows2_v, ssem):
        wid = lax.axis_index("s") * nc + lax.axis_index("c")
        base = wid * rows_per_w
        ld1 = pltpu.async_copy(idx_hbm.at[pl.ds(base, rows_per_w)], idx_v, ssem)
        ld2 = pltpu.async_copy(t_hbm, table_v, ssem)
        ld1.wait()
        ld2.wait()
        half = OUT_DIM // 32
        stores = []
        for j in range(n_chunks):
            @plsc.parallel_loop(0, 8, unroll=4)
            def body(it, j=j):
                i0 = it * 16
                idxv = idx_v[pl.ds(j * 128 + i0, 16)]
                vs = [idxv[u] for u in range(16)]

                def loads(v):
                    return [table_v[v, pl.ds(k * 16, 16)]
                            for k in range(OUT_DIM // 16)]

                cur = loads(vs[0])
                for u in range(16):
                    nxt = loads(vs[u + 1]) if u < 15 else None
                    r = it * 8 + u // 2
                    c = (u % 2) * N_LATENTS
                    for k in range(half):
                        rows1_v[j, r, pl.ds(c + k * 16, 16)] = cur[k]
                        rows2_v[j, r, pl.ds(c + k * 16, 16)] = cur[half + k]
                    cur = nxt
            stores.append(pltpu.async_copy(
                rows1_v.at[j],
                out1_hbm.at[pl.ds(base // 2 + j * 64, 64)], ssem))
            stores.append(pltpu.async_copy(
                rows2_v.at[j],
                out2_hbm.at[pl.ds(base // 2 + j * 64, 64)], ssem))
        for s in stores:
            s.wait()

    return expand


def kernel(x, emb, gamma, beta, W, b):
    x = x.astype(jnp.int32)
    (t_cat,) = _table_call(
        x, emb, gamma.reshape(1, EMB_DIM),
        beta.reshape(1, EMB_DIM), W, b.reshape(1, OUT_DIM))
    expand = _make_expand()
    o1p, o2p = expand(x, t_cat)
    return (o1p.reshape(BATCH, N_LATENTS), o2p.reshape(BATCH, N_LATENTS))


# speedup vs baseline: 1.0574x; 1.0574x over previous
import functools

import jax
import jax.numpy as jnp
from jax import lax
from jax.experimental import pallas as pl
from jax.experimental.pallas import tpu as pltpu
from jax.experimental.pallas import tpu_sc as plsc

N_LATENTS = 64
BATCH = 16384
VOCAB = 10
EMB_DIM = 50
OUT_DIM = 2 * N_LATENTS
EPS = 1e-5


def _table_body(x_ref, emb_ref, gamma_ref, beta_ref, w_ref, b_ref, t_ref):
    xv = x_ref[...]
    iota = lax.broadcasted_iota(jnp.int32, (VOCAB, BATCH), 0)
    onehot = (xv[None, :] == iota).astype(jnp.float32)
    counts = jnp.sum(onehot, axis=1).reshape(1, VOCAB)

    emb = emb_ref[...]
    mean = lax.dot_general(counts, emb, (((1,), (0,)), ((), ())),
                           precision=lax.Precision.HIGHEST) * (1.0 / BATCH)
    centered = emb - mean
    var = lax.dot_general(counts, centered * centered, (((1,), (0,)), ((), ())),
                          precision=lax.Precision.HIGHEST) * (1.0 / BATCH)
    inv = 1.0 / jnp.sqrt(var + EPS)
    tbl = jnp.maximum(centered * (inv * gamma_ref[...]) + beta_ref[...], 0.0)
    y = lax.dot_general(tbl, w_ref[...], (((1,), (1,)), ((), ())),
                        precision=lax.Precision.HIGHEST) + b_ref[...]
    t_ref[...] = y


_table_call = pl.pallas_call(
    _table_body,
    out_shape=[jax.ShapeDtypeStruct((VOCAB, OUT_DIM), jnp.float32)],
)


def _split_body(y_ref, o1_ref, o2_ref):
    yv = y_ref[...]
    o1_ref[...] = yv[:, :N_LATENTS]
    o2_ref[...] = yv[:, N_LATENTS:]


_SPLIT_BLK = 4096
_split_call = pl.pallas_call(
    _split_body,
    grid=(BATCH // _SPLIT_BLK,),
    in_specs=[pl.BlockSpec((_SPLIT_BLK, OUT_DIM), lambda i: (i, 0))],
    out_specs=[
        pl.BlockSpec((_SPLIT_BLK, N_LATENTS), lambda i: (i, 0)),
        pl.BlockSpec((_SPLIT_BLK, N_LATENTS), lambda i: (i, 0)),
    ],
    out_shape=[
        jax.ShapeDtypeStruct((BATCH, N_LATENTS), jnp.float32),
        jax.ShapeDtypeStruct((BATCH, N_LATENTS), jnp.float32),
    ],
)


def _make_expand():
    info = plsc.get_sparse_core_info()
    nc, ns = info.num_cores, info.num_subcores
    nw = nc * ns
    rows_per_w = BATCH // nw
    n_chunks = rows_per_w // 128

    mesh = plsc.VectorSubcoreMesh(core_axis_name="c", subcore_axis_name="s")

    @functools.partial(
        pl.kernel,
        mesh=mesh,
        out_type=[
            jax.ShapeDtypeStruct((BATCH // 2, OUT_DIM), jnp.float32),
            jax.ShapeDtypeStruct((BATCH // 2, OUT_DIM), jnp.float32),
        ],
        scratch_types=[
            pltpu.VMEM((rows_per_w,), jnp.int32),
            pltpu.VMEM((VOCAB, OUT_DIM), jnp.float32),
            pltpu.VMEM((n_chunks, 64, OUT_DIM), jnp.float32),
            pltpu.VMEM((n_chunks, 64, OUT_DIM), jnp.float32),
            pltpu.SemaphoreType.DMA,
        ],
        compiler_params=pltpu.CompilerParams(use_tc_tiling_on_sc=False),
    )
    def expand(idx_hbm, t_hbm, out1_hbm, out2_hbm,
               idx_v, table_v, rows1_v, rows2_v, ssem):
        wid = lax.axis_index("s") * nc + lax.axis_index("c")
        base = wid * rows_per_w
        ld1 = pltpu.async_copy(idx_hbm.at[pl.ds(base, rows_per_w)], idx_v, ssem)
        ld2 = pltpu.async_copy(t_hbm, table_v, ssem)
        ld1.wait()
        ld2.wait()
        half = OUT_DIM // 32
        stores = []
        for j in range(n_chunks):
            @plsc.parallel_loop(0, 8, unroll=1)
            def body(it, j=j):
                i0 = it * 16
                idxv = idx_v[pl.ds(j * 128 + i0, 16)]
                vs = [idxv[u] for u in range(16)]

                def loads(v):
                    return [table_v[v, pl.ds(k * 16, 16)]
                            for k in range(OUT_DIM // 16)]

                cur = loads(vs[0])
                for u in range(16):
                    nxt = loads(vs[u + 1]) if u < 15 else None
                    r = it * 8 + u // 2
                    c = (u % 2) * N_LATENTS
                    for k in range(half):
                        rows1_v[j, r, pl.ds(c + k * 16, 16)] = cur[k]
                        rows2_v[j, r, pl.ds(c + k * 16, 16)] = cur[half + k]
                    cur = nxt
            stores.append(pltpu.async_copy(
                rows1_v.at[j],
                out1_hbm.at[pl.ds(base // 2 + j * 64, 64)], ssem))
            stores.append(pltpu.async_copy(
                rows2_v.at[j],
                out2_hbm.at[pl.ds(base // 2 + j * 64, 64)], ssem))
        for s in stores:
            s.wait()

    return expand


def kernel(x, emb, gamma, beta, W, b):
    x = x.astype(jnp.int32)
    (t_cat,) = _table_call(
        x, emb, gamma.reshape(1, EMB_DIM),
        beta.reshape(1, EMB_DIM), W, b.reshape(1, OUT_DIM))
    expand = _make_expand()
    o1p, o2p = expand(x, t_cat)
    return (o1p.reshape(BATCH, N_LATENTS), o2p.reshape(BATCH, N_LATENTS))


# speedup vs baseline: 1.0899x; 1.0307x over previous
import functools

import jax
import jax.numpy as jnp
from jax import lax
from jax.experimental import pallas as pl
from jax.experimental.pallas import tpu as pltpu
from jax.experimental.pallas import tpu_sc as plsc

N_LATENTS = 64
BATCH = 16384
VOCAB = 10
EMB_DIM = 50
OUT_DIM = 2 * N_LATENTS
EPS = 1e-5


def _table_body(x_ref, emb_ref, gamma_ref, beta_ref, w_ref, b_ref, t_ref):
    xv = x_ref[...]
    iota = lax.broadcasted_iota(jnp.int32, (VOCAB, BATCH), 0)
    onehot = (xv[None, :] == iota).astype(jnp.float32)
    counts = jnp.sum(onehot, axis=1).reshape(1, VOCAB)

    emb = emb_ref[...]
    mean = lax.dot_general(counts, emb, (((1,), (0,)), ((), ())),
                           precision=lax.Precision.HIGHEST) * (1.0 / BATCH)
    centered = emb - mean
    var = lax.dot_general(counts, centered * centered, (((1,), (0,)), ((), ())),
                          precision=lax.Precision.HIGHEST) * (1.0 / BATCH)
    inv = 1.0 / jnp.sqrt(var + EPS)
    tbl = jnp.maximum(centered * (inv * gamma_ref[...]) + beta_ref[...], 0.0)
    y = lax.dot_general(tbl, w_ref[...], (((1,), (1,)), ((), ())),
                        precision=lax.Precision.HIGHEST) + b_ref[...]
    t_ref[...] = y


_table_call = pl.pallas_call(
    _table_body,
    out_shape=[jax.ShapeDtypeStruct((VOCAB, OUT_DIM), jnp.float32)],
)


def _split_body(y_ref, o1_ref, o2_ref):
    yv = y_ref[...]
    o1_ref[...] = yv[:, :N_LATENTS]
    o2_ref[...] = yv[:, N_LATENTS:]


_SPLIT_BLK = 4096
_split_call = pl.pallas_call(
    _split_body,
    grid=(BATCH // _SPLIT_BLK,),
    in_specs=[pl.BlockSpec((_SPLIT_BLK, OUT_DIM), lambda i: (i, 0))],
    out_specs=[
        pl.BlockSpec((_SPLIT_BLK, N_LATENTS), lambda i: (i, 0)),
        pl.BlockSpec((_SPLIT_BLK, N_LATENTS), lambda i: (i, 0)),
    ],
    out_shape=[
        jax.ShapeDtypeStruct((BATCH, N_LATENTS), jnp.float32),
        jax.ShapeDtypeStruct((BATCH, N_LATENTS), jnp.float32),
    ],
)


def _make_expand():
    info = plsc.get_sparse_core_info()
    nc, ns = info.num_cores, info.num_subcores
    nw = nc * ns
    rows_per_w = BATCH // nw
    n_chunks = rows_per_w // 128

    mesh = plsc.VectorSubcoreMesh(core_axis_name="c", subcore_axis_name="s")

    @functools.partial(
        pl.kernel,
        mesh=mesh,
        out_type=[
            jax.ShapeDtypeStruct((BATCH // 2, OUT_DIM), jnp.float32),
            jax.ShapeDtypeStruct((BATCH // 2, OUT_DIM), jnp.float32),
        ],
        scratch_types=[
            pltpu.VMEM((rows_per_w,), jnp.int32),
            pltpu.VMEM((VOCAB, OUT_DIM), jnp.float32),
            pltpu.VMEM((n_chunks, 64, OUT_DIM), jnp.float32),
            pltpu.VMEM((n_chunks, 64, OUT_DIM), jnp.float32),
            pltpu.SemaphoreType.DMA,
        ],
        compiler_params=pltpu.CompilerParams(use_tc_tiling_on_sc=False),
    )
    def expand(idx_hbm, t_hbm, out1_hbm, out2_hbm,
               idx_v, table_v, rows1_v, rows2_v, ssem):
        wid = lax.axis_index("s") * nc + lax.axis_index("c")
        base = wid * rows_per_w
        ld1 = pltpu.async_copy(idx_hbm.at[pl.ds(base, rows_per_w)], idx_v, ssem)
        ld2 = pltpu.async_copy(t_hbm, table_v, ssem)
        ld1.wait()
        ld2.wait()
        half = OUT_DIM // 32
        stores = []
        for j in range(n_chunks):
            @plsc.parallel_loop(0, 8, unroll=2)
            def body(it, j=j):
                i0 = it * 16
                idxv = idx_v[pl.ds(j * 128 + i0, 16)]
                vs = [idxv[u] for u in range(16)]

                def loads(v):
                    return [table_v[v, pl.ds(k * 16, 16)]
                            for k in range(OUT_DIM // 16)]

                pipe = [loads(vs[0]), loads(vs[1])]
                for u in range(16):
                    if u < 14:
                        pipe.append(loads(vs[u + 2]))
                    cur = pipe[u]
                    r = it * 8 + u // 2
                    c = (u % 2) * N_LATENTS
                    for k in range(half):
                        rows1_v[j, r, pl.ds(c + k * 16, 16)] = cur[k]
                        rows2_v[j, r, pl.ds(c + k * 16, 16)] = cur[half + k]
            stores.append(pltpu.async_copy(
                rows1_v.at[j],
                out1_hbm.at[pl.ds(base // 2 + j * 64, 64)], ssem))
            stores.append(pltpu.async_copy(
                rows2_v.at[j],
                out2_hbm.at[pl.ds(base // 2 + j * 64, 64)], ssem))
        for s in stores:
            s.wait()

    return expand


def kernel(x, emb, gamma, beta, W, b):
    x = x.astype(jnp.int32)
    (t_cat,) = _table_call(
        x, emb, gamma.reshape(1, EMB_DIM),
        beta.reshape(1, EMB_DIM), W, b.reshape(1, OUT_DIM))
    expand = _make_expand()
    o1p, o2p = expand(x, t_cat)
    return (o1p.reshape(BATCH, N_LATENTS), o2p.reshape(BATCH, N_LATENTS))
